# baseline (speedup 1.0000x reference)
import functools

import jax
import jax.numpy as jnp
from jax import lax
from jax.experimental import pallas as pl
from jax.experimental.pallas import tpu as pltpu
from jax.experimental.pallas import tpu_sc as plsc

N = 10000
T = 12
IN = 128
H = 128
OUT = 128
E = 320000

NC = 2
NS = 16
NW = NC * NS
C = 64
QN = 4
QC = 40
K = QN * QC
E_PAD = NW * K * C
NROW = 632
NPAD = NS * NROW
DEGW = 8

BLK = 400
NBLK = N // BLK


def _sc_aggregate(x2d, src_all, dst_resh, zrows):
    mesh = plsc.VectorSubcoreMesh(core_axis_name="c", subcore_axis_name="s")
    out_type = jax.ShapeDtypeStruct((NC, T + 1, NPAD, H), jnp.float32)
    scratch = [
        pltpu.VMEM((QC, C), jnp.int32),
        pltpu.VMEM((QC, C), jnp.int32),
        [pltpu.VMEM((C, H), jnp.float32) for _ in range(4)],
        [pltpu.SemaphoreType.DMA for _ in range(4)],
        [pltpu.SemaphoreType.DMA for _ in range(4)],
    ]
    scratch.append(pltpu.VMEM_SHARED((NPAD, H), jnp.float32))

    @functools.partial(pl.kernel, out_type=out_type, mesh=mesh,
                       scratch_types=scratch)
    def body(x_hbm, srcall_hbm, dst_hbm, z_hbm, agg_hbm,
             srcv, dstv, bufs, gsems, ssems, acc):
        cid = lax.axis_index("c")
        sid = lax.axis_index("s")
        w = cid * NS + sid

        def gstart(c, b):
            pltpu.make_async_copy(x_hbm.at[srcv.at[c]], bufs[b],
                                  gsems[b]).start()

        def gwait(c, b):
            pltpu.make_async_copy(x_hbm.at[srcv.at[c]], bufs[b],
                                  gsems[b]).wait()

        def sstart(c, b):
            pltpu.async_copy(bufs[b], acc.at[dstv.at[c]], ssems[b], add=True)

        def swait(c, b):
            pltpu.make_async_copy(bufs[b], acc.at[dstv.at[c]],
                                  ssems[b]).wait()

        QG = QC // 2

        def tstep(t, _):
            pltpu.sync_copy(z_hbm, acc.at[pl.ds(sid * NROW, NROW)])
            plsc.subcore_barrier()

            def quarter(q, _1):
                pltpu.sync_copy(srcall_hbm.at[t, w, q], srcv)
                pltpu.sync_copy(dst_hbm.at[w, q], dstv)
                def gbody(g, _2):
                    c0 = 4 * g
                    for b in range(4):
                        gstart(c0 + b, b)
                    for b in range(4):
                        gwait(c0 + b, b)
                        sstart(c0 + b, b)
                    for b in range(4):
                        swait(c0 + b, b)
                    return _2
                lax.fori_loop(0, QC // 4, gbody, 0)
                return _1
            lax.fori_loop(0, QN, quarter, 0)

            plsc.subcore_barrier()
            pltpu.sync_copy(acc.at[pl.ds(sid * NROW, NROW)],
                            agg_hbm.at[cid, t, pl.ds(sid * NROW, NROW)])
            plsc.subcore_barrier()
            return _
        lax.fori_loop(0, T + 1, tstep, 0)

    return body(x2d, src_all, dst_resh, zrows)


def _tc_body(x_ref, agg_ref,
             W_l_ref, b_l_ref, W_r_ref, W_ihT_ref, W_hhT_ref, b_ih_ref,
             b_hh_ref, W_rec_ref, b_rec_ref, W_c1_ref, b_c1_ref, W_c2r_ref,
             b_c2_ref, recon_ref, cls_ref):
    deg = agg_ref[0, T, :, 0] + agg_ref[1, T, :, 0]
    inv_deg = (1.0 / jnp.maximum(deg, 1.0))[:, None]
    W_l = W_l_ref[...]
    W_r = W_r_ref[...]
    W_ihT = W_ihT_ref[...]
    W_hhT = W_hhT_ref[...]
    b_ih = b_ih_ref[...]
    b_hh = b_hh_ref[...]
    W_rec = W_rec_ref[...]
    b_rec = b_rec_ref[...]
    W_c1 = W_c1_ref[...]
    b_c1 = b_c1_ref[...]
    W_c2r = W_c2r_ref[...]
    b_c2 = b_c2_ref[0, 0]
    b_l = b_l_ref[...]

    h = jnp.zeros((BLK, H), jnp.float32)
    for t in range(T):
        x_t = x_ref[:, t, :]
        agg = (agg_ref[0, t] + agg_ref[1, t]) * inv_deg
        spatial = jax.nn.relu(
            jnp.dot(agg, W_l, preferred_element_type=jnp.float32)
            + jnp.dot(x_t, W_r, preferred_element_type=jnp.float32) + b_l)
        gi = jnp.dot(spatial, W_ihT, preferred_element_type=jnp.float32) + b_ih
        gh = jnp.dot(h, W_hhT, preferred_element_type=jnp.float32) + b_hh
        r = jax.nn.sigmoid(gi[:, :H] + gh[:, :H])
        z = jax.nn.sigmoid(gi[:, H:2 * H] + gh[:, H:2 * H])
        n = jnp.tanh(gi[:, 2 * H:] + r * gh[:, 2 * H:])
        h = (1.0 - z) * n + z * h
        recon_ref[:, t, :] = (
            jnp.dot(h, W_rec, preferred_element_type=jnp.float32) + b_rec)
        c1 = jax.nn.relu(
            jnp.dot(h, W_c1, preferred_element_type=jnp.float32) + b_c1)
        logit = jnp.sum(c1 * W_c2r, axis=1) + b_c2
        cls_ref[:, t:t + 1] = jax.nn.sigmoid(logit)[:, None]


def _tc_dense(x, agg2, W_l, b_l, W_r, W_ihT, W_hhT, b_ih, b_hh,
              W_rec, b_rec, W_c1, b_c1, W_c2r, b_c2):
    full = lambda shape: pl.BlockSpec(shape, lambda i: tuple(0 for _ in shape))
    grid_spec = pl.GridSpec(
        grid=(NBLK,),
        in_specs=[
            pl.BlockSpec((BLK, T, IN), lambda i: (i, 0, 0)),
            pl.BlockSpec((NC, T + 1, BLK, H), lambda i: (0, 0, i, 0)),
            full((IN, H)), full((1, H)), full((IN, H)),
            full((H, 3 * H)), full((H, 3 * H)), full((1, 3 * H)),
            full((1, 3 * H)),
            full((H, OUT)), full((1, OUT)),
            full((H, H // 2)), full((1, H // 2)), full((1, H // 2)),
            full((1, 1)),
        ],
        out_specs=[
            pl.BlockSpec((BLK, T, OUT), lambda i: (i, 0, 0)),
            pl.BlockSpec((BLK, T), lambda i: (i, 0)),
        ],
    )
    return pl.pallas_call(
        _tc_body,
        grid_spec=grid_spec,
        out_shape=[
            jax.ShapeDtypeStruct((N, T, OUT), jnp.float32),
            jax.ShapeDtypeStruct((N, T), jnp.float32),
        ],
    )(x, agg2, W_l, b_l, W_r, W_ihT, W_hhT, b_ih, b_hh,
      W_rec, b_rec, W_c1, b_c1, W_c2r, b_c2)


def kernel(x, edge_index, W_l, b_l, W_r, W_ih, W_hh, b_ih, b_hh,
           W_rec, b_rec, W_c1, b_c1, W_c2, b_c2):
    src = edge_index[0]
    dst = edge_index[1]

    pad = E_PAD - E
    src_p = jnp.concatenate([src, jnp.zeros((pad,), jnp.int32)])
    dst_p = jnp.concatenate([dst, jnp.full((pad,), N, jnp.int32)])

    t_off = jnp.arange(T, dtype=jnp.int32)[:, None]
    src_t = src_p[None, :] * T + t_off
    src_deg = jnp.full((1, E_PAD), N * T, jnp.int32)
    src_all = jnp.concatenate([src_t, src_deg], axis=0).reshape(
        T + 1, NW, QN, QC, C)
    dst_resh = dst_p.reshape(NW, QN, QC, C)
    x2d = jnp.concatenate(
        [x.reshape(N * T, IN), jnp.ones((8, IN), jnp.float32)], axis=0)

    zrows = jnp.zeros((NROW, H), jnp.float32)
    agg2 = _sc_aggregate(x2d, src_all, dst_resh, zrows)

    recon, cls = _tc_dense(
        x, agg2,
        W_l, b_l.reshape(1, H), W_r,
        W_ih.T, W_hh.T, b_ih.reshape(1, 3 * H), b_hh.reshape(1, 3 * H),
        W_rec, b_rec.reshape(1, OUT),
        W_c1, b_c1.reshape(1, H // 2), W_c2.reshape(1, H // 2),
        b_c2.reshape(1, 1))
    return (recon, cls)


# speedup vs baseline: 2.2196x; 2.2196x over previous
import functools

import jax
import jax.numpy as jnp
from jax import lax
from jax.experimental import pallas as pl
from jax.experimental.pallas import tpu as pltpu
from jax.experimental.pallas import tpu_sc as plsc

N = 10000
T = 12
IN = 128
H = 128
OUT = 128
E = 320000

NC = 2
NS = 16
NW = NC * NS
C = 64
QN = 4
QC = 40
K = QN * QC
E_PAD = NW * K * C
NROW = 632
NPAD = NS * NROW
DEGW = 8

BLK = 400
NBLK = N // BLK


def _sc_aggregate(x2d, src_all, dst_resh, zrows):
    mesh = plsc.VectorSubcoreMesh(core_axis_name="c", subcore_axis_name="s")
    out_type = jax.ShapeDtypeStruct((NC, T + 1, NPAD, H), jnp.float32)
    scratch = [
        pltpu.VMEM((QC, C), jnp.int32),
        pltpu.VMEM((QC, C), jnp.int32),
        [pltpu.VMEM((C, H), jnp.float32) for _ in range(4)],
        [pltpu.SemaphoreType.DMA for _ in range(4)],
        [pltpu.SemaphoreType.DMA for _ in range(4)],
    ]
    scratch.append(pltpu.VMEM_SHARED((NPAD, H), jnp.float32))

    @functools.partial(pl.kernel, out_type=out_type, mesh=mesh,
                       scratch_types=scratch)
    def body(x_hbm, srcall_hbm, dst_hbm, z_hbm, agg_hbm,
             srcv, dstv, bufs, gsems, ssems, acc):
        cid = lax.axis_index("c")
        sid = lax.axis_index("s")
        w = cid * NS + sid

        def gstart(c, b):
            pltpu.make_async_copy(x_hbm.at[srcv.at[c]], bufs[b],
                                  gsems[b]).start()

        def gwait(c, b):
            pltpu.make_async_copy(x_hbm.at[srcv.at[c]], bufs[b],
                                  gsems[b]).wait()

        def sstart(c, b):
            pltpu.async_copy(bufs[b], acc.at[dstv.at[c]], ssems[b], add=True)

        def swait(c, b):
            pltpu.make_async_copy(bufs[b], acc.at[dstv.at[c]],
                                  ssems[b]).wait()

        QG = QC // 2

        def tstep(t, _):
            pltpu.sync_copy(z_hbm, acc.at[pl.ds(sid * NROW, NROW)])
            plsc.subcore_barrier()

            def quarter(q, _1):
                pltpu.sync_copy(srcall_hbm.at[t, w, q], srcv)
                pltpu.sync_copy(dst_hbm.at[w, q], dstv)
                def gbody(g, _2):
                    c0 = 4 * g
                    for b in range(4):
                        gstart(c0 + b, b)
                    for b in range(4):
                        gwait(c0 + b, b)
                        sstart(c0 + b, b)
                    for b in range(4):
                        swait(c0 + b, b)
                    return _2
                lax.fori_loop(0, QC // 4, gbody, 0)
                return _1
            lax.fori_loop(0, QN, quarter, 0)

            plsc.subcore_barrier()
            pltpu.sync_copy(acc.at[pl.ds(sid * NROW, NROW)],
                            agg_hbm.at[cid, t, pl.ds(sid * NROW, NROW)])
            plsc.subcore_barrier()
            return _
        lax.fori_loop(0, T, tstep, 0)

        pltpu.sync_copy(z_hbm, acc.at[pl.ds(sid * NROW, NROW)])
        pltpu.sync_copy(srcall_hbm.at[T, w, 0], srcv)
        plsc.subcore_barrier()
        gstart(0, 0)
        gwait(0, 0)

        def deg_quarter(q, _1):
            pltpu.sync_copy(dst_hbm.at[w, q], dstv)

            def dchunk(j, _2):
                c0 = 4 * j
                for b in range(4):
                    pltpu.async_copy(bufs[0], acc.at[dstv.at[c0 + b]],
                                     ssems[b], add=True)
                for b in range(4):
                    pltpu.make_async_copy(bufs[0], acc.at[dstv.at[c0 + b]],
                                          ssems[b]).wait()
                return _2
            lax.fori_loop(0, QC // 4, dchunk, 0)
            return _1
        lax.fori_loop(0, QN, deg_quarter, 0)

        plsc.subcore_barrier()
        pltpu.sync_copy(acc.at[pl.ds(sid * NROW, NROW)],
                        agg_hbm.at[cid, T, pl.ds(sid * NROW, NROW)])
        plsc.subcore_barrier()

    return body(x2d, src_all, dst_resh, zrows)


def _tc_body(x_ref, agg_ref,
             W_l_ref, b_l_ref, W_r_ref, W_ihT_ref, W_hhT_ref, b_ih_ref,
             b_hh_ref, W_rec_ref, b_rec_ref, W_c1_ref, b_c1_ref, W_c2r_ref,
             b_c2_ref, recon_ref, cls_ref):
    deg = agg_ref[0, T, :, 0] + agg_ref[1, T, :, 0]
    inv_deg = (1.0 / jnp.maximum(deg, 1.0))[:, None]
    W_l = W_l_ref[...]
    W_r = W_r_ref[...]
    W_ihT = W_ihT_ref[...]
    W_hhT = W_hhT_ref[...]
    b_ih = b_ih_ref[...]
    b_hh = b_hh_ref[...]
    W_rec = W_rec_ref[...]
    b_rec = b_rec_ref[...]
    W_c1 = W_c1_ref[...]
    b_c1 = b_c1_ref[...]
    W_c2r = W_c2r_ref[...]
    b_c2 = b_c2_ref[0, 0]
    b_l = b_l_ref[...]

    h = jnp.zeros((BLK, H), jnp.float32)
    for t in range(T):
        x_t = x_ref[:, t, :]
        agg = (agg_ref[0, t] + agg_ref[1, t]) * inv_deg
        spatial = jax.nn.relu(
            jnp.dot(agg, W_l, preferred_element_type=jnp.float32)
            + jnp.dot(x_t, W_r, preferred_element_type=jnp.float32) + b_l)
        gi = jnp.dot(spatial, W_ihT, preferred_element_type=jnp.float32) + b_ih
        gh = jnp.dot(h, W_hhT, preferred_element_type=jnp.float32) + b_hh
        r = jax.nn.sigmoid(gi[:, :H] + gh[:, :H])
        z = jax.nn.sigmoid(gi[:, H:2 * H] + gh[:, H:2 * H])
        n = jnp.tanh(gi[:, 2 * H:] + r * gh[:, 2 * H:])
        h = (1.0 - z) * n + z * h
        recon_ref[:, t, :] = (
            jnp.dot(h, W_rec, preferred_element_type=jnp.float32) + b_rec)
        c1 = jax.nn.relu(
            jnp.dot(h, W_c1, preferred_element_type=jnp.float32) + b_c1)
        logit = jnp.sum(c1 * W_c2r, axis=1) + b_c2
        cls_ref[:, t:t + 1] = jax.nn.sigmoid(logit)[:, None]


def _tc_dense(x, agg2, W_l, b_l, W_r, W_ihT, W_hhT, b_ih, b_hh,
              W_rec, b_rec, W_c1, b_c1, W_c2r, b_c2):
    full = lambda shape: pl.BlockSpec(shape, lambda i: tuple(0 for _ in shape))
    grid_spec = pl.GridSpec(
        grid=(NBLK,),
        in_specs=[
            pl.BlockSpec((BLK, T, IN), lambda i: (i, 0, 0)),
            pl.BlockSpec((NC, T + 1, BLK, H), lambda i: (0, 0, i, 0)),
            full((IN, H)), full((1, H)), full((IN, H)),
            full((H, 3 * H)), full((H, 3 * H)), full((1, 3 * H)),
            full((1, 3 * H)),
            full((H, OUT)), full((1, OUT)),
            full((H, H // 2)), full((1, H // 2)), full((1, H // 2)),
            full((1, 1)),
        ],
        out_specs=[
            pl.BlockSpec((BLK, T, OUT), lambda i: (i, 0, 0)),
            pl.BlockSpec((BLK, T), lambda i: (i, 0)),
        ],
    )
    return pl.pallas_call(
        _tc_body,
        grid_spec=grid_spec,
        out_shape=[
            jax.ShapeDtypeStruct((N, T, OUT), jnp.float32),
            jax.ShapeDtypeStruct((N, T), jnp.float32),
        ],
    )(x, agg2, W_l, b_l, W_r, W_ihT, W_hhT, b_ih, b_hh,
      W_rec, b_rec, W_c1, b_c1, W_c2r, b_c2)


def kernel(x, edge_index, W_l, b_l, W_r, W_ih, W_hh, b_ih, b_hh,
           W_rec, b_rec, W_c1, b_c1, W_c2, b_c2):
    src = edge_index[0]
    dst = edge_index[1]

    pad = E_PAD - E
    src_p = jnp.concatenate([src, jnp.zeros((pad,), jnp.int32)])
    dst_p = jnp.concatenate([dst, jnp.full((pad,), N, jnp.int32)])

    t_off = jnp.arange(T, dtype=jnp.int32)[:, None]
    src_t = src_p[None, :] * T + t_off
    src_deg = jnp.full((1, E_PAD), N * T, jnp.int32)
    src_all = jnp.concatenate([src_t, src_deg], axis=0).reshape(
        T + 1, NW, QN, QC, C)
    dst_resh = dst_p.reshape(NW, QN, QC, C)
    x2d = jnp.concatenate(
        [x.reshape(N * T, IN), jnp.ones((8, IN), jnp.float32)], axis=0)

    zrows = jnp.zeros((NROW, H), jnp.float32)
    agg2 = _sc_aggregate(x2d, src_all, dst_resh, zrows)

    recon, cls = _tc_dense(
        x, agg2,
        W_l, b_l.reshape(1, H), W_r,
        W_ih.T, W_hh.T, b_ih.reshape(1, 3 * H), b_hh.reshape(1, 3 * H),
        W_rec, b_rec.reshape(1, OUT),
        W_c1, b_c1.reshape(1, H // 2), W_c2.reshape(1, H // 2),
        b_c2.reshape(1, 1))
    return (recon, cls)


# speedup vs baseline: 2.3103x; 1.0409x over previous
import functools

import jax
import jax.numpy as jnp
from jax import lax
from jax.experimental import pallas as pl
from jax.experimental.pallas import tpu as pltpu
from jax.experimental.pallas import tpu_sc as plsc

N = 10000
T = 12
IN = 128
H = 128
OUT = 128
E = 320000

NC = 2
NS = 16
NW = NC * NS
C = 64
QN = 4
QC = 40
K = QN * QC
E_PAD = NW * K * C
NROW = 632
NPAD = NS * NROW
DEGW = 8

BLK = 400
NBLK = N // BLK


def _sc_aggregate(x2d, src_all, dst_resh, zrows):
    mesh = plsc.VectorSubcoreMesh(core_axis_name="c", subcore_axis_name="s")
    out_type = jax.ShapeDtypeStruct((NC, T + 1, NPAD, H), jnp.float32)
    scratch = [
        pltpu.VMEM((QC, C), jnp.int32),
        pltpu.VMEM((QC, C), jnp.int32),
        [pltpu.VMEM((C, H), jnp.float32) for _ in range(4)],
        [pltpu.SemaphoreType.DMA for _ in range(4)],
        [pltpu.SemaphoreType.DMA for _ in range(4)],
    ]
    scratch.append(pltpu.VMEM_SHARED((NPAD, H), jnp.float32))

    @functools.partial(pl.kernel, out_type=out_type, mesh=mesh,
                       scratch_types=scratch)
    def body(x_hbm, srcall_hbm, dst_hbm, z_hbm, agg_hbm,
             srcv, dstv, bufs, gsems, ssems, acc):
        cid = lax.axis_index("c")
        sid = lax.axis_index("s")
        w = cid * NS + sid

        def gstart(c, b):
            pltpu.make_async_copy(x_hbm.at[srcv.at[c]], bufs[b],
                                  gsems[b]).start()

        def gwait(c, b):
            pltpu.make_async_copy(x_hbm.at[srcv.at[c]], bufs[b],
                                  gsems[b]).wait()

        def sstart(c, b):
            pltpu.async_copy(bufs[b], acc.at[dstv.at[c]], ssems[b], add=True)

        def swait(c, b):
            pltpu.make_async_copy(bufs[b], acc.at[dstv.at[c]],
                                  ssems[b]).wait()

        QG = QC // 2

        def tstep(t, _):
            pltpu.sync_copy(z_hbm, acc.at[pl.ds(sid * NROW, NROW)])
            plsc.subcore_barrier()

            def quarter(q, _1):
                pltpu.sync_copy(srcall_hbm.at[t, w, q], srcv)
                pltpu.sync_copy(dst_hbm.at[w, q], dstv)
                gstart(0, 0)
                gstart(1, 1)

                def gbody(gg, _2):
                    for sub in (0, 1):
                        g = 2 * gg + sub
                        b0, b1 = (0, 1) if sub == 0 else (2, 3)
                        o0, o1 = (2, 3) if sub == 0 else (0, 1)
                        c0 = 2 * g
                        gwait(c0, b0)
                        gwait(c0 + 1, b1)
                        sstart(c0, b0)
                        sstart(c0 + 1, b1)

                        @pl.when(g + 1 < QG)
                        def _next():
                            @pl.when(g >= 1)
                            def _drain_prev():
                                swait(2 * (g - 1), o0)
                                swait(2 * (g - 1) + 1, o1)
                            gstart(2 * (g + 1), o0)
                            gstart(2 * (g + 1) + 1, o1)
                    return _2
                lax.fori_loop(0, QG // 2, gbody, 0)
                swait(2 * (QG - 2), 0)
                swait(2 * (QG - 2) + 1, 1)
                swait(2 * (QG - 1), 2)
                swait(2 * (QG - 1) + 1, 3)
                return _1
            lax.fori_loop(0, QN, quarter, 0)

            plsc.subcore_barrier()
            pltpu.sync_copy(acc.at[pl.ds(sid * NROW, NROW)],
                            agg_hbm.at[cid, t, pl.ds(sid * NROW, NROW)])
            plsc.subcore_barrier()
            return _
        lax.fori_loop(0, T, tstep, 0)

        pltpu.sync_copy(z_hbm, acc.at[pl.ds(sid * NROW, NROW)])
        pltpu.sync_copy(srcall_hbm.at[T, w, 0], srcv)
        plsc.subcore_barrier()
        gstart(0, 0)
        gwait(0, 0)

        def deg_quarter(q, _1):
            pltpu.sync_copy(dst_hbm.at[w, q], dstv)

            def dchunk(j, _2):
                c0 = 4 * j
                for b in range(4):
                    pltpu.async_copy(bufs[0], acc.at[dstv.at[c0 + b]],
                                     ssems[b], add=True)
                for b in range(4):
                    pltpu.make_async_copy(bufs[0], acc.at[dstv.at[c0 + b]],
                                          ssems[b]).wait()
                return _2
            lax.fori_loop(0, QC // 4, dchunk, 0)
            return _1
        lax.fori_loop(0, QN, deg_quarter, 0)

        plsc.subcore_barrier()
        pltpu.sync_copy(acc.at[pl.ds(sid * NROW, NROW)],
                        agg_hbm.at[cid, T, pl.ds(sid * NROW, NROW)])
        plsc.subcore_barrier()

    return body(x2d, src_all, dst_resh, zrows)


def _tc_body(x_ref, agg_ref,
             W_l_ref, b_l_ref, W_r_ref, W_ihT_ref, W_hhT_ref, b_ih_ref,
             b_hh_ref, W_rec_ref, b_rec_ref, W_c1_ref, b_c1_ref, W_c2r_ref,
             b_c2_ref, recon_ref, cls_ref):
    deg = agg_ref[0, T, :, 0] + agg_ref[1, T, :, 0]
    inv_deg = (1.0 / jnp.maximum(deg, 1.0))[:, None]
    W_l = W_l_ref[...]
    W_r = W_r_ref[...]
    W_ihT = W_ihT_ref[...]
    W_hhT = W_hhT_ref[...]
    b_ih = b_ih_ref[...]
    b_hh = b_hh_ref[...]
    W_rec = W_rec_ref[...]
    b_rec = b_rec_ref[...]
    W_c1 = W_c1_ref[...]
    b_c1 = b_c1_ref[...]
    W_c2r = W_c2r_ref[...]
    b_c2 = b_c2_ref[0, 0]
    b_l = b_l_ref[...]

    h = jnp.zeros((BLK, H), jnp.float32)
    for t in range(T):
        x_t = x_ref[:, t, :]
        agg = (agg_ref[0, t] + agg_ref[1, t]) * inv_deg
        spatial = jax.nn.relu(
            jnp.dot(agg, W_l, preferred_element_type=jnp.float32)
            + jnp.dot(x_t, W_r, preferred_element_type=jnp.float32) + b_l)
        gi = jnp.dot(spatial, W_ihT, preferred_element_type=jnp.float32) + b_ih
        gh = jnp.dot(h, W_hhT, preferred_element_type=jnp.float32) + b_hh
        r = jax.nn.sigmoid(gi[:, :H] + gh[:, :H])
        z = jax.nn.sigmoid(gi[:, H:2 * H] + gh[:, H:2 * H])
        n = jnp.tanh(gi[:, 2 * H:] + r * gh[:, 2 * H:])
        h = (1.0 - z) * n + z * h
        recon_ref[:, t, :] = (
            jnp.dot(h, W_rec, preferred_element_type=jnp.float32) + b_rec)
        c1 = jax.nn.relu(
            jnp.dot(h, W_c1, preferred_element_type=jnp.float32) + b_c1)
        logit = jnp.sum(c1 * W_c2r, axis=1) + b_c2
        cls_ref[:, t:t + 1] = jax.nn.sigmoid(logit)[:, None]


def _tc_dense(x, agg2, W_l, b_l, W_r, W_ihT, W_hhT, b_ih, b_hh,
              W_rec, b_rec, W_c1, b_c1, W_c2r, b_c2):
    full = lambda shape: pl.BlockSpec(shape, lambda i: tuple(0 for _ in shape))
    grid_spec = pl.GridSpec(
        grid=(NBLK,),
        in_specs=[
            pl.BlockSpec((BLK, T, IN), lambda i: (i, 0, 0)),
            pl.BlockSpec((NC, T + 1, BLK, H), lambda i: (0, 0, i, 0)),
            full((IN, H)), full((1, H)), full((IN, H)),
            full((H, 3 * H)), full((H, 3 * H)), full((1, 3 * H)),
            full((1, 3 * H)),
            full((H, OUT)), full((1, OUT)),
            full((H, H // 2)), full((1, H // 2)), full((1, H // 2)),
            full((1, 1)),
        ],
        out_specs=[
            pl.BlockSpec((BLK, T, OUT), lambda i: (i, 0, 0)),
            pl.BlockSpec((BLK, T), lambda i: (i, 0)),
        ],
    )
    return pl.pallas_call(
        _tc_body,
        grid_spec=grid_spec,
        out_shape=[
            jax.ShapeDtypeStruct((N, T, OUT), jnp.float32),
            jax.ShapeDtypeStruct((N, T), jnp.float32),
        ],
    )(x, agg2, W_l, b_l, W_r, W_ihT, W_hhT, b_ih, b_hh,
      W_rec, b_rec, W_c1, b_c1, W_c2r, b_c2)


def kernel(x, edge_index, W_l, b_l, W_r, W_ih, W_hh, b_ih, b_hh,
           W_rec, b_rec, W_c1, b_c1, W_c2, b_c2):
    src = edge_index[0]
    dst = edge_index[1]

    pad = E_PAD - E
    src_p = jnp.concatenate([src, jnp.zeros((pad,), jnp.int32)])
    dst_p = jnp.concatenate([dst, jnp.full((pad,), N, jnp.int32)])

    t_off = jnp.arange(T, dtype=jnp.int32)[:, None]
    src_t = src_p[None, :] * T + t_off
    src_deg = jnp.full((1, E_PAD), N * T, jnp.int32)
    src_all = jnp.concatenate([src_t, src_deg], axis=0).reshape(
        T + 1, NW, QN, QC, C)
    dst_resh = dst_p.reshape(NW, QN, QC, C)
    x2d = jnp.concatenate(
        [x.reshape(N * T, IN), jnp.ones((8, IN), jnp.float32)], axis=0)

    zrows = jnp.zeros((NROW, H), jnp.float32)
    agg2 = _sc_aggregate(x2d, src_all, dst_resh, zrows)

    recon, cls = _tc_dense(
        x, agg2,
        W_l, b_l.reshape(1, H), W_r,
        W_ih.T, W_hh.T, b_ih.reshape(1, 3 * H), b_hh.reshape(1, 3 * H),
        W_rec, b_rec.reshape(1, OUT),
        W_c1, b_c1.reshape(1, H // 2), W_c2.reshape(1, H // 2),
        b_c2.reshape(1, 1))
    return (recon, cls)
